# baseline (speedup 1.0000x reference)
import dataclasses

import jax
import jax.numpy as jnp
from jax import lax
from jax.experimental import pallas as pl
from jax.experimental.pallas import tpu as pltpu
from jax.experimental.pallas import tpu_sc as plsc

T = 26
V = 100000
D = 16
B = 16384
NC, NS = 2, 16
NW = NC * NS
BW = B // NW
IW = 128
KC = BW // IW
NU = T * KC
RING = 8
NT = (T * B) // IW


def _gather_body(keys_hbm, tbl_hbm, out_hbm, kbuf, rbuf, tbuf, ksem, gsem,
                 osem):
    cid = lax.axis_index("core")
    sid = lax.axis_index("subcore")
    wid = sid * NC + cid

    pltpu.sync_copy(keys_hbm.at[:, wid], kbuf)

    lanes = lax.iota(jnp.int32, 16)

    def gather_desc(u, slot):
        t = u // KC
        return pltpu.make_async_copy(
            tbl_hbm.at[t].at[kbuf.at[t, u % KC]], rbuf.at[slot], gsem.at[slot])

    def out_desc(u, slot, half):
        t = u // KC
        tile = t * (B // IW) + wid * KC + (u % KC)
        return pltpu.make_async_copy(
            tbuf.at[slot, pl.ds(half * 8, 8), pl.ds(0, IW)],
            out_hbm.at[half, tile], osem.at[slot])

    for b in range(RING):
        gather_desc(b, b).start()

    @pl.loop(0, NU, step=RING)
    def _group(g):
        for b in range(RING):
            u = g + b
            gather_desc(u, b).wait()

            @pl.when(u >= RING)
            def _():
                out_desc(u - RING, b, 0).wait()
                out_desc(u - RING, b, 1).wait()

            for l in range(IW):
                plsc.store_scatter(tbuf.at[b],
                                   [lanes, jnp.full((16,), l, jnp.int32)],
                                   rbuf[b, l])

            out_desc(u, b, 0).start()
            out_desc(u, b, 1).start()

            @pl.when(u + RING < NU)
            def _():
                gather_desc(u + RING, b).start()

    for b in range(RING):
        out_desc(b, b, 0).wait()
        out_desc(b, b, 1).wait()


def _compiler_params():
    cp = pltpu.CompilerParams(use_tc_tiling_on_sc=False)
    if "needs_layout_passes" in pltpu.CompilerParams.__dataclass_fields__:
        cp = dataclasses.replace(cp, needs_layout_passes=False)
    return cp


def kernel(keys_list, tables):
    keys_r = keys_list.reshape(T, NW, KC, IW).astype(jnp.int32)
    mesh = plsc.VectorSubcoreMesh(core_axis_name="core",
                                  subcore_axis_name="subcore")
    out = pl.kernel(
        _gather_body,
        out_type=jax.ShapeDtypeStruct((2, NT, 8, IW), jnp.float32),
        mesh=mesh,
        compiler_params=_compiler_params(),
        scratch_types=[
            pltpu.VMEM((T, KC, IW), jnp.int32),
            pltpu.VMEM((RING, IW, D), jnp.float32),
            pltpu.VMEM((RING, D, 129), jnp.float32),
            pltpu.SemaphoreType.DMA,
            pltpu.SemaphoreType.DMA((RING,)),
            pltpu.SemaphoreType.DMA((RING,)),
        ],
    )(keys_r, tables)
    return out.transpose(1, 3, 0, 2).reshape(T * B, D)


# speedup vs baseline: 1.0069x; 1.0069x over previous
import dataclasses

import jax
import jax.numpy as jnp
from jax import lax
from jax.experimental import pallas as pl
from jax.experimental.pallas import tpu as pltpu
from jax.experimental.pallas import tpu_sc as plsc

T = 26
V = 100000
D = 16
B = 16384
NC, NS = 2, 16
NW = NC * NS
BW = B // NW
IW = 128
KC = BW // IW
NU = T * KC
RING = 4
NT = (T * B) // IW


def _gather_body(keys_hbm, tbl_hbm, out_hbm, kbuf, rbuf, tbuf, ksem, gsem,
                 osem):
    cid = lax.axis_index("core")
    sid = lax.axis_index("subcore")
    wid = sid * NC + cid

    pltpu.sync_copy(keys_hbm.at[:, wid], kbuf)

    lanes = lax.iota(jnp.int32, 16)

    def gather_desc(u, slot):
        t = u // KC
        return pltpu.make_async_copy(
            tbl_hbm.at[t].at[kbuf.at[t, u % KC]], rbuf.at[slot], gsem.at[slot])

    def out_desc(u, slot, half):
        t = u // KC
        tile = t * (B // IW) + wid * KC + (u % KC)
        return pltpu.make_async_copy(
            tbuf.at[slot, pl.ds(half * 8, 8), pl.ds(0, IW)],
            out_hbm.at[half, tile], osem.at[slot])

    for b in range(RING):
        gather_desc(b, b).start()

    @pl.loop(0, NU, step=RING)
    def _group(g):
        for b in range(RING):
            u = g + b
            gather_desc(u, b).wait()

            @pl.when(u >= RING)
            def _():
                out_desc(u - RING, b, 0).wait()
                out_desc(u - RING, b, 1).wait()

            for l in range(IW):
                plsc.store_scatter(tbuf.at[b],
                                   [lanes, jnp.full((16,), l, jnp.int32)],
                                   rbuf[b, l])

            out_desc(u, b, 0).start()
            out_desc(u, b, 1).start()

            @pl.when(u + RING < NU)
            def _():
                gather_desc(u + RING, b).start()

    for b in range(RING):
        out_desc(b, b, 0).wait()
        out_desc(b, b, 1).wait()


def _compiler_params():
    cp = pltpu.CompilerParams(use_tc_tiling_on_sc=False)
    if "needs_layout_passes" in pltpu.CompilerParams.__dataclass_fields__:
        cp = dataclasses.replace(cp, needs_layout_passes=False)
    return cp


def kernel(keys_list, tables):
    keys_r = keys_list.reshape(T, NW, KC, IW).astype(jnp.int32)
    mesh = plsc.VectorSubcoreMesh(core_axis_name="core",
                                  subcore_axis_name="subcore")
    out = pl.kernel(
        _gather_body,
        out_type=jax.ShapeDtypeStruct((2, NT, 8, IW), jnp.float32),
        mesh=mesh,
        compiler_params=_compiler_params(),
        scratch_types=[
            pltpu.VMEM((T, KC, IW), jnp.int32),
            pltpu.VMEM((RING, IW, D), jnp.float32),
            pltpu.VMEM((RING, D, 129), jnp.float32),
            pltpu.SemaphoreType.DMA,
            pltpu.SemaphoreType.DMA((RING,)),
            pltpu.SemaphoreType.DMA((RING,)),
        ],
    )(keys_r, tables)
    return out.transpose(1, 3, 0, 2).reshape(T * B, D)
